# baseline (speedup 1.0000x reference)
import jax
import jax.numpy as jnp
from jax.experimental import pallas as pl
from jax.experimental.pallas import tpu as pltpu

_TOP_K = 8
_SQRT_HALF = 0.7071067811865476
_LN_EPS = 1e-5


def _epilogue(acc, b1, gamma, beta, w2, b2):
    h = acc + b1
    mu = jnp.mean(h, axis=-1, keepdims=True)
    xc = h - mu
    var = jnp.mean(xc * xc, axis=-1, keepdims=True)
    h = xc * jax.lax.rsqrt(var + _LN_EPS) * gamma + beta
    h = 0.5 * h * (1.0 + jax.lax.erf(h * _SQRT_HALF))
    logits = jnp.dot(h, w2, preferred_element_type=jnp.float32) + b2

    e_dim = logits.shape[-1]
    iota_e = jax.lax.broadcasted_iota(jnp.int32, logits.shape, 1)
    rank = jnp.zeros(logits.shape, jnp.int32)
    for f in range(e_dim):
        lf = logits[:, f : f + 1]
        beats = (lf > logits) | ((lf == logits) & (iota_e > f))
        rank = rank + beats.astype(jnp.int32)
    sel = rank < _TOP_K

    m = jnp.max(logits, axis=-1, keepdims=True)
    ex = jnp.where(sel, jnp.exp(logits - m), 0.0)
    s = jnp.sum(ex, axis=-1, keepdims=True)
    return ex / s


def _body(x_ref, w1_ref, b1_ref, g_ref, bt_ref, w2_ref, b2_ref, o_ref):
    x = x_ref[...].astype(jnp.bfloat16)
    acc = jnp.dot(x[:, 0, :], w1_ref[0], preferred_element_type=jnp.float32)
    acc += jnp.dot(x[:, 1, :], w1_ref[1], preferred_element_type=jnp.float32)
    acc += jnp.dot(x[:, 2, :], w1_ref[2], preferred_element_type=jnp.float32)
    o_ref[...] = _epilogue(
        acc, b1_ref[...], g_ref[...], bt_ref[...], w2_ref[...], b2_ref[...]
    )


def kernel(triplet, W1, b1, gamma, beta, W2, b2):
    b_dim, three, d_in = triplet.shape
    d_out = W1.shape[1]
    e_dim = W2.shape[1]

    bm = min(128, b_dim)
    nb = b_dim // bm

    w1s = W1.reshape(three, d_in, d_out).astype(jnp.bfloat16)
    b1r = b1.reshape(1, d_out)
    gr = gamma.reshape(1, d_out)
    btr = beta.reshape(1, d_out)
    b2r = b2.reshape(1, e_dim)

    out = pl.pallas_call(
        _body,
        grid=(nb,),
        in_specs=[
            pl.BlockSpec((bm, three, d_in), lambda i: (i, 0, 0)),
            pl.BlockSpec((three, d_in, d_out), lambda i: (0, 0, 0)),
            pl.BlockSpec((1, d_out), lambda i: (0, 0)),
            pl.BlockSpec((1, d_out), lambda i: (0, 0)),
            pl.BlockSpec((1, d_out), lambda i: (0, 0)),
            pl.BlockSpec((d_out, e_dim), lambda i: (0, 0)),
            pl.BlockSpec((1, e_dim), lambda i: (0, 0)),
        ],
        out_specs=pl.BlockSpec((bm, e_dim), lambda i: (i, 0)),
        out_shape=jax.ShapeDtypeStruct((b_dim, e_dim), jnp.float32),
        compiler_params=pltpu.CompilerParams(
            dimension_semantics=("arbitrary",)
        ),
    )(triplet, w1s, b1r, gr, btr, W2, b2r)
    return out
